# baseline (speedup 1.0000x reference)
import jax
import jax.numpy as jnp
from jax.experimental import pallas as pl
from jax.experimental.pallas import tpu as pltpu

HID = 64
NV = HID // 2
NS = HID // 2
EV = HID // 2
ES = HID // 2
LAYERS = 2

_F32 = jnp.float32


def _gather_rows(onehot_t, x):
    return jax.lax.dot_general(
        onehot_t, x, (((0,), (0,)), ((), ())), preferred_element_type=_F32)


def _gvp(p, v, s, relu_s, sig_v):
    Wh = p['Wh']
    Ws = p['Ws']
    bs = p['bs']
    Wv = p['Wv']
    vh = [jnp.dot(vc, Wh, preferred_element_type=_F32) for vc in v]
    vn = jnp.sqrt(vh[0] * vh[0] + vh[1] * vh[1] + vh[2] * vh[2] + 1e-8)
    so = jnp.dot(jnp.concatenate([s, vn], axis=-1), Ws,
                 preferred_element_type=_F32) + bs
    if relu_s:
        so = jnp.maximum(so, 0.0)
    vo = [jnp.dot(vc, Wv, preferred_element_type=_F32) for vc in vh]
    if sig_v:
        g = jax.nn.sigmoid(
            jnp.sqrt(vo[0] * vo[0] + vo[1] * vo[1] + vo[2] * vo[2] + 1e-8))
        vo = [vc * g for vc in vo]
    return vo, so


def _gvp_norm(v, s):
    vn2 = v[0] * v[0] + v[1] * v[1] + v[2] * v[2]
    sigma = jnp.sqrt(jnp.mean(vn2, axis=-1, keepdims=True) + 1e-8)
    v = [vc / sigma for vc in v]
    mu = jnp.mean(s, axis=-1, keepdims=True)
    var = jnp.mean((s - mu) * (s - mu), axis=-1, keepdims=True)
    s = (s - mu) / jnp.sqrt(var + 1e-8)
    return v, s


def _split_vs(x, nv):
    v = [x[:, i * nv:(i + 1) * nv] for i in range(3)]
    s = x[:, 3 * nv:]
    return v, s


def _merge_vs(v, s):
    return jnp.concatenate(v + [s], axis=-1)


def _term_kernel(N, K, metas, treedef, V_ref, E_ref, idx_ref, mask_ref,
                 pk_ref, ohc_ref, hV_out, hE_out):
    NK = N * K
    leaves = [pk_ref[off:off + r, :c] for (off, r, c) in metas]
    p_ref = jax.tree_util.tree_unflatten(treedef, leaves)
    Vt = V_ref[0]
    Et = E_ref[0]
    idx = idx_ref[0]
    mask_col = mask_ref[0]

    src_iota = jax.lax.broadcasted_iota(jnp.int32, (N, NK), 0)
    oh_j = (src_iota == idx).astype(_F32)
    oh_exp = ohc_ref[:N]
    fsel_t = ohc_ref[N:]
    oh_n0 = jax.lax.dot_general(
        oh_j, fsel_t, (((1,), (1,)), ((), ())),
        preferred_element_type=_F32)
    oh_i = jnp.dot(oh_n0, oh_exp, preferred_element_type=_F32)

    vV, sV = _split_vs(Vt, NV)
    hv_v, hv_s = _gvp(p_ref['W_v'], vV, sV, False, False)
    vE, sE = _split_vs(Et, EV)
    he_v, he_s = _gvp(p_ref['W_e'], vE, sE, False, False)

    mask_exp = _gather_rows(oh_exp, mask_col)
    ma = _gather_rows(oh_j, mask_col)
    mask_attend = mask_exp * ma

    for li in range(LAYERS):
        pn = p_ref['node%d' % li]
        pe = p_ref['edge%d' % li]

        hv_m = _merge_vs(hv_v, hv_s)
        g_i = _gather_rows(oh_i, hv_m)
        g_j = _gather_rows(oh_j, hv_m)
        g_x = _gather_rows(oh_exp, hv_m)
        gi_v, gi_s = _split_vs(g_i, NV)
        gj_v, gj_s = _split_vs(g_j, NV)
        gx_v, gx_s = _split_vs(g_x, NV)
        h_v = [jnp.concatenate([gx_v[c], gi_v[c], gj_v[c], he_v[c]], axis=-1)
               for c in range(3)]
        h_s = jnp.concatenate([gx_s, gi_s, gj_s, he_s], axis=-1)
        m_v, m_s = _gvp(pn['g1'], h_v, h_s, True, True)
        m_v, m_s = _gvp(pn['g2'], m_v, m_s, True, True)
        m_v, m_s = _gvp(pn['g3'], m_v, m_s, False, False)
        m = _merge_vs(m_v, m_s) * mask_attend
        dh = jnp.dot(oh_exp, m, preferred_element_type=_F32) * (1.0 / K)
        dh_v, dh_s = _split_vs(dh, NV)
        hv_v = [hv_v[c] + dh_v[c] for c in range(3)]
        hv_s = hv_s + dh_s
        hv_v, hv_s = _gvp_norm(hv_v, hv_s)
        d_v, d_s = _gvp(pn['ff1'], hv_v, hv_s, True, True)
        d_v, d_s = _gvp(pn['ff2'], d_v, d_s, False, False)
        hv_v = [hv_v[c] + d_v[c] for c in range(3)]
        hv_s = hv_s + d_s
        hv_v, hv_s = _gvp_norm(hv_v, hv_s)
        hv_v = [vc * mask_col for vc in hv_v]
        hv_s = hv_s * mask_col

        hv_m = _merge_vs(hv_v, hv_s)
        g_i = _gather_rows(oh_i, hv_m)
        g_j = _gather_rows(oh_j, hv_m)
        gi_v, gi_s = _split_vs(g_i, NV)
        gj_v, gj_s = _split_vs(g_j, NV)
        h_v = [jnp.concatenate([gi_v[c], gj_v[c], he_v[c]], axis=-1)
               for c in range(3)]
        h_s = jnp.concatenate([gi_s, gj_s, he_s], axis=-1)
        m_v, m_s = _gvp(pe['g1'], h_v, h_s, True, True)
        m_v, m_s = _gvp(pe['g2'], m_v, m_s, True, True)
        m_v, m_s = _gvp(pe['g3'], m_v, m_s, False, False)
        he_v = [he_v[c] + m_v[c] * mask_attend for c in range(3)]
        he_s = he_s + m_s * mask_attend
        he_v, he_s = _gvp_norm(he_v, he_s)
        d_v, d_s = _gvp(pe['ff1'], he_v, he_s, True, True)
        d_v, d_s = _gvp(pe['ff2'], d_v, d_s, False, False)
        he_v = [he_v[c] + d_v[c] for c in range(3)]
        he_s = he_s + d_s
        he_v, he_s = _gvp_norm(he_v, he_s)
        he_v = [vc * mask_attend for vc in he_v]
        he_s = he_s * mask_attend

    ho_v, ho_s = _gvp(p_ref['W_out'], he_v, he_s, False, False)
    hV_out[0] = _merge_vs(hv_v, hv_s)
    hE_out[0] = _merge_vs(ho_v, ho_s)


def kernel(V, E, E_idx, mask, params):
    B, T, N, K = E_idx.shape
    BT = B * T
    CV = V.shape[-1]
    CE = E.shape[-1]
    NK = N * K

    Vr = V.reshape(BT, N, CV)
    Er = E.reshape(BT, NK, CE)
    idxr = E_idx.reshape(BT, 1, NK).astype(jnp.int32)
    maskr = mask.reshape(BT, N, 1).astype(_F32)

    p2 = jax.tree_util.tree_map(
        lambda a: a.reshape(1, -1) if a.ndim == 1 else a, params)
    leaves, treedef = jax.tree_util.tree_flatten(p2)
    metas = []
    chunks = []
    off = 0
    for a in leaves:
        r, c = a.shape
        rp = -(-r // 8) * 8
        metas.append((off, r, c))
        pad = jnp.zeros((rp, 128), _F32)
        chunks.append(pad.at[:r, :c].set(a.astype(_F32)))
        off += rp
    packed = jnp.concatenate(chunks, axis=0)

    n_iota = jnp.arange(N, dtype=jnp.int32)[:, None]
    e_iota = jnp.arange(NK, dtype=jnp.int32)[None, :]
    oh_exp_c = (n_iota == e_iota // K).astype(_F32)
    fsel_t_c = (e_iota == n_iota * K).astype(_F32)
    oh_consts = jnp.concatenate([oh_exp_c, fsel_t_c], axis=0)

    import functools
    body = functools.partial(_term_kernel, N, K, tuple(metas), treedef)

    hV, hE = pl.pallas_call(
        body,
        grid=(BT,),
        in_specs=[
            pl.BlockSpec((1, N, CV), lambda i: (i, 0, 0)),
            pl.BlockSpec((1, NK, CE), lambda i: (i, 0, 0)),
            pl.BlockSpec((1, 1, NK), lambda i: (i, 0, 0)),
            pl.BlockSpec((1, N, 1), lambda i: (i, 0, 0)),
            pl.BlockSpec(packed.shape, lambda i: (0, 0)),
            pl.BlockSpec(oh_consts.shape, lambda i: (0, 0)),
        ],
        out_specs=[
            pl.BlockSpec((1, N, 3 * NV + NS), lambda i: (i, 0, 0)),
            pl.BlockSpec((1, NK, 3 * EV + ES), lambda i: (i, 0, 0)),
        ],
        out_shape=[
            jax.ShapeDtypeStruct((BT, N, 3 * NV + NS), _F32),
            jax.ShapeDtypeStruct((BT, NK, 3 * EV + ES), _F32),
        ],
    )(Vr, Er, idxr, maskr, packed, oh_consts)

    return (hV.reshape(B, T, N, 3 * NV + NS),
            hE.reshape(B, T, N, K, 3 * EV + ES))


# speedup vs baseline: 1.0936x; 1.0936x over previous
import jax
import jax.numpy as jnp
from jax.experimental import pallas as pl
from jax.experimental.pallas import tpu as pltpu

HID = 64
NV = HID // 2
NS = HID // 2
EV = HID // 2
ES = HID // 2
LAYERS = 2

_F32 = jnp.float32


def _gather_rows(onehot_t, x):
    return jax.lax.dot_general(
        onehot_t, x, (((0,), (0,)), ((), ())), preferred_element_type=_F32)


def _gvp(p, v, s, relu_s, sig_v):
    Wh = p['Wh']
    Ws = p['Ws']
    bs = p['bs']
    Wv = p['Wv']
    vh = [jnp.dot(vc, Wh, preferred_element_type=_F32) for vc in v]
    vn = jnp.sqrt(vh[0] * vh[0] + vh[1] * vh[1] + vh[2] * vh[2] + 1e-8)
    so = jnp.dot(jnp.concatenate([s, vn], axis=-1), Ws,
                 preferred_element_type=_F32) + bs
    if relu_s:
        so = jnp.maximum(so, 0.0)
    vo = [jnp.dot(vc, Wv, preferred_element_type=_F32) for vc in vh]
    if sig_v:
        g = jax.nn.sigmoid(
            jnp.sqrt(vo[0] * vo[0] + vo[1] * vo[1] + vo[2] * vo[2] + 1e-8))
        vo = [vc * g for vc in vo]
    return vo, so


def _g1_fused(p, oh_cat, srcs_v, srcs_s, loc_v, loc_s):
    Wh = p['Wh']
    Ws = p['Ws']
    bs = p['bs']
    Wv = p['Wv']
    ns = len(srcs_v) + 1
    vh = []
    for c in range(3):
        pcat = jnp.concatenate(
            [jnp.dot(sv[c], Wh[32 * i:32 * (i + 1)],
                     preferred_element_type=_F32)
             for i, sv in enumerate(srcs_v)], axis=0)
        vh.append(_gather_rows(oh_cat, pcat)
                  + jnp.dot(loc_v[c], Wh[32 * (ns - 1):32 * ns],
                            preferred_element_type=_F32))
    vn = jnp.sqrt(vh[0] * vh[0] + vh[1] * vh[1] + vh[2] * vh[2] + 1e-8)
    qcat = jnp.concatenate(
        [jnp.dot(ss, Ws[32 * i:32 * (i + 1)], preferred_element_type=_F32)
         for i, ss in enumerate(srcs_s)], axis=0)
    so = (_gather_rows(oh_cat, qcat)
          + jnp.dot(loc_s, Ws[32 * (ns - 1):32 * ns],
                    preferred_element_type=_F32)
          + jnp.dot(vn, Ws[32 * ns:], preferred_element_type=_F32) + bs)
    so = jnp.maximum(so, 0.0)
    vo = [jnp.dot(vc, Wv, preferred_element_type=_F32) for vc in vh]
    g = jax.nn.sigmoid(
        jnp.sqrt(vo[0] * vo[0] + vo[1] * vo[1] + vo[2] * vo[2] + 1e-8))
    return [vc * g for vc in vo], so


def _gvp_norm(v, s):
    vn2 = v[0] * v[0] + v[1] * v[1] + v[2] * v[2]
    sigma = jnp.sqrt(jnp.mean(vn2, axis=-1, keepdims=True) + 1e-8)
    v = [vc / sigma for vc in v]
    mu = jnp.mean(s, axis=-1, keepdims=True)
    var = jnp.mean((s - mu) * (s - mu), axis=-1, keepdims=True)
    s = (s - mu) / jnp.sqrt(var + 1e-8)
    return v, s


def _split_vs(x, nv):
    v = [x[:, i * nv:(i + 1) * nv] for i in range(3)]
    s = x[:, 3 * nv:]
    return v, s


def _merge_vs(v, s):
    return jnp.concatenate(v + [s], axis=-1)


def _term_kernel(N, K, metas, treedef, V_ref, E_ref, idx_ref, mask_ref,
                 pk_ref, hV_out, hE_out):
    NK = N * K
    leaves = [pk_ref[off:off + r, :c] for (off, r, c) in metas]
    p_ref = jax.tree_util.tree_unflatten(treedef, leaves)
    Vt = V_ref[0]
    Et = E_ref[0]
    idx = idx_ref[0]
    mask_col = mask_ref[0]

    src_iota = jax.lax.broadcasted_iota(jnp.int32, (N, NK), 0)
    edge_iota = jax.lax.broadcasted_iota(jnp.int32, (N, NK), 1)
    oh_j = (src_iota == idx).astype(_F32)
    oh_exp = (src_iota == edge_iota // K).astype(_F32)
    fsel = (jax.lax.broadcasted_iota(jnp.int32, (NK, N), 0) ==
            jax.lax.broadcasted_iota(jnp.int32, (NK, N), 1) * K).astype(_F32)
    oh_n0 = jnp.dot(oh_j, fsel, preferred_element_type=_F32)
    oh_i = jnp.dot(oh_n0, oh_exp, preferred_element_type=_F32)

    vV, sV = _split_vs(Vt, NV)
    hv_v, hv_s = _gvp(p_ref['W_v'], vV, sV, False, False)
    vE, sE = _split_vs(Et, EV)
    he_v, he_s = _gvp(p_ref['W_e'], vE, sE, False, False)

    mask_exp = _gather_rows(oh_exp, mask_col)
    ma = _gather_rows(oh_j, mask_col)
    mask_attend = mask_exp * ma

    oh3 = jnp.concatenate([oh_exp, oh_i, oh_j], axis=0)
    oh2 = oh3[N:]

    for li in range(LAYERS):
        pn = p_ref['node%d' % li]
        pe = p_ref['edge%d' % li]

        m_v, m_s = _g1_fused(pn['g1'], oh3, [hv_v] * 3, [hv_s] * 3,
                             he_v, he_s)
        m_v, m_s = _gvp(pn['g2'], m_v, m_s, True, True)
        m_v, m_s = _gvp(pn['g3'], m_v, m_s, False, False)
        m = _merge_vs(m_v, m_s) * mask_attend
        dh = jnp.dot(oh_exp, m, preferred_element_type=_F32) * (1.0 / K)
        dh_v, dh_s = _split_vs(dh, NV)
        hv_v = [hv_v[c] + dh_v[c] for c in range(3)]
        hv_s = hv_s + dh_s
        hv_v, hv_s = _gvp_norm(hv_v, hv_s)
        d_v, d_s = _gvp(pn['ff1'], hv_v, hv_s, True, True)
        d_v, d_s = _gvp(pn['ff2'], d_v, d_s, False, False)
        hv_v = [hv_v[c] + d_v[c] for c in range(3)]
        hv_s = hv_s + d_s
        hv_v, hv_s = _gvp_norm(hv_v, hv_s)
        hv_v = [vc * mask_col for vc in hv_v]
        hv_s = hv_s * mask_col

        m_v, m_s = _g1_fused(pe['g1'], oh2, [hv_v] * 2, [hv_s] * 2,
                             he_v, he_s)
        m_v, m_s = _gvp(pe['g2'], m_v, m_s, True, True)
        m_v, m_s = _gvp(pe['g3'], m_v, m_s, False, False)
        he_v = [he_v[c] + m_v[c] * mask_attend for c in range(3)]
        he_s = he_s + m_s * mask_attend
        he_v, he_s = _gvp_norm(he_v, he_s)
        d_v, d_s = _gvp(pe['ff1'], he_v, he_s, True, True)
        d_v, d_s = _gvp(pe['ff2'], d_v, d_s, False, False)
        he_v = [he_v[c] + d_v[c] for c in range(3)]
        he_s = he_s + d_s
        he_v, he_s = _gvp_norm(he_v, he_s)
        he_v = [vc * mask_attend for vc in he_v]
        he_s = he_s * mask_attend

    ho_v, ho_s = _gvp(p_ref['W_out'], he_v, he_s, False, False)
    hV_out[0] = _merge_vs(hv_v, hv_s)
    hE_out[0] = _merge_vs(ho_v, ho_s)


def kernel(V, E, E_idx, mask, params):
    B, T, N, K = E_idx.shape
    BT = B * T
    CV = V.shape[-1]
    CE = E.shape[-1]
    NK = N * K

    Vr = V.reshape(BT, N, CV)
    Er = E.reshape(BT, NK, CE)
    idxr = E_idx.reshape(BT, 1, NK).astype(jnp.int32)
    maskr = mask.reshape(BT, N, 1).astype(_F32)

    p2 = jax.tree_util.tree_map(
        lambda a: a.reshape(1, -1) if a.ndim == 1 else a, params)
    leaves, treedef = jax.tree_util.tree_flatten(p2)
    metas = []
    chunks = []
    off = 0
    for a in leaves:
        r, c = a.shape
        rp = -(-r // 8) * 8
        metas.append((off, r, c))
        pad = jnp.zeros((rp, 128), _F32)
        chunks.append(pad.at[:r, :c].set(a.astype(_F32)))
        off += rp
    packed = jnp.concatenate(chunks, axis=0)

    import functools
    body = functools.partial(_term_kernel, N, K, tuple(metas), treedef)

    hV, hE = pl.pallas_call(
        body,
        grid=(BT,),
        in_specs=[
            pl.BlockSpec((1, N, CV), lambda i: (i, 0, 0)),
            pl.BlockSpec((1, NK, CE), lambda i: (i, 0, 0)),
            pl.BlockSpec((1, 1, NK), lambda i: (i, 0, 0)),
            pl.BlockSpec((1, N, 1), lambda i: (i, 0, 0)),
            pl.BlockSpec(packed.shape, lambda i: (0, 0)),
        ],
        out_specs=[
            pl.BlockSpec((1, N, 3 * NV + NS), lambda i: (i, 0, 0)),
            pl.BlockSpec((1, NK, 3 * EV + ES), lambda i: (i, 0, 0)),
        ],
        out_shape=[
            jax.ShapeDtypeStruct((BT, N, 3 * NV + NS), _F32),
            jax.ShapeDtypeStruct((BT, NK, 3 * EV + ES), _F32),
        ],
    )(Vr, Er, idxr, maskr, packed)

    return (hV.reshape(B, T, N, 3 * NV + NS),
            hE.reshape(B, T, N, K, 3 * EV + ES))
